# baseline (speedup 1.0000x reference)
import functools

import jax
import jax.numpy as jnp
from jax import lax
from jax.experimental import pallas as pl
from jax.experimental.pallas import tpu as pltpu
from jax.experimental.pallas import tpu_sc as plsc

BATCH = 16384
DIM = 64
NC = 2
NS = 16
NW = NC * NS
BPW = BATCH // NW
ICH = 128
NCH = BPW // ICH
L = 16
NG = BPW // L


def _rsqrt16(x):
    i = plsc.bitcast(x, jnp.int32)
    i = jnp.int32(0x5F3759DF) - lax.shift_right_arithmetic(i, jnp.int32(1))
    y = plsc.bitcast(i, jnp.float32)
    xh = x * jnp.float32(0.5)
    for _ in range(3):
        y = y * (jnp.float32(1.5) - xh * y * y)
    return y


def _tile_kernel(heads, rels, tails, ent, rel, out,
                 ih, ir, it, hr, rr, tr, sc, sem):
    wid = lax.axis_index("s") * NC + lax.axis_index("c")
    base = wid * BPW

    for c in range(NCH):
        off = base + c * ICH
        pltpu.sync_copy(heads.at[pl.ds(off, ICH)], ih.at[c])
        pltpu.sync_copy(rels.at[pl.ds(off, ICH)], ir.at[c])
        pltpu.sync_copy(tails.at[pl.ds(off, ICH)], it.at[c])

    copies = []
    for c in range(NCH):
        rows = pl.ds(c * ICH, ICH)
        copies.append(pltpu.async_copy(ent.at[ih.at[c]], hr.at[rows, :], sem))
        copies.append(pltpu.async_copy(rel.at[ir.at[c]], rr.at[rows, :], sem))
        copies.append(pltpu.async_copy(ent.at[it.at[c]], tr.at[rows, :], sem))
    for cp in copies:
        cp.wait()

    iota = lax.iota(jnp.int32, L)
    zero = jnp.zeros((L,), jnp.float32)

    def group(g, _):
        rowi = g * L + iota

        sh0 = zero
        sh1 = zero
        st0 = zero
        st1 = zero
        for j in range(0, DIM, 2):
            c0 = jnp.full((L,), j, jnp.int32)
            c1 = jnp.full((L,), j + 1, jnp.int32)
            hv0 = plsc.load_gather(hr, [rowi, c0])
            hv1 = plsc.load_gather(hr, [rowi, c1])
            tv0 = plsc.load_gather(tr, [rowi, c0])
            tv1 = plsc.load_gather(tr, [rowi, c1])
            sh0 = sh0 + hv0 * hv0
            sh1 = sh1 + hv1 * hv1
            st0 = st0 + tv0 * tv0
            st1 = st1 + tv1 * tv1
        invh = _rsqrt16(sh0 + sh1)
        invt = _rsqrt16(st0 + st1)

        a0 = zero
        a1 = zero
        for j in range(0, DIM, 2):
            c0 = jnp.full((L,), j, jnp.int32)
            c1 = jnp.full((L,), j + 1, jnp.int32)
            hv0 = plsc.load_gather(hr, [rowi, c0])
            hv1 = plsc.load_gather(hr, [rowi, c1])
            tv0 = plsc.load_gather(tr, [rowi, c0])
            tv1 = plsc.load_gather(tr, [rowi, c1])
            rv0 = plsc.load_gather(rr, [rowi, c0])
            rv1 = plsc.load_gather(rr, [rowi, c1])
            a0 = a0 + jnp.abs(hv0 * invh + rv0 - tv0 * invt)
            a1 = a1 + jnp.abs(hv1 * invh + rv1 - tv1 * invt)
        sc[pl.ds(g * L, L)] = a0 + a1
        return 0

    lax.fori_loop(0, NG, group, 0)

    pltpu.sync_copy(sc, out.at[pl.ds(base, BPW)])


@jax.jit
def _transe_sc(heads, rels, tails, ent, rel):
    mesh = plsc.VectorSubcoreMesh(core_axis_name="c", subcore_axis_name="s")
    f = functools.partial(
        pl.kernel,
        mesh=mesh,
        out_type=jax.ShapeDtypeStruct((BATCH,), jnp.float32),
        scratch_types=[
            pltpu.VMEM((NCH, ICH), jnp.int32),
            pltpu.VMEM((NCH, ICH), jnp.int32),
            pltpu.VMEM((NCH, ICH), jnp.int32),
            pltpu.VMEM((BPW, DIM), jnp.float32),
            pltpu.VMEM((BPW, DIM), jnp.float32),
            pltpu.VMEM((BPW, DIM), jnp.float32),
            pltpu.VMEM((BPW,), jnp.float32),
            pltpu.SemaphoreType.DMA,
        ],
        compiler_params=pltpu.CompilerParams(
            use_tc_tiling_on_sc=False,
            needs_layout_passes=False,
        ),
    )(_tile_kernel)
    return f(heads, rels, tails, ent, rel)


def kernel(heads, relations, tails, entity_emb, relation_emb):
    heads = jnp.asarray(heads, jnp.int32)
    relations = jnp.asarray(relations, jnp.int32)
    tails = jnp.asarray(tails, jnp.int32)
    return _transe_sc(heads, relations, tails, entity_emb, relation_emb)
